# Initial kernel scaffold; baseline (speedup 1.0000x reference)
import jax
import jax.numpy as jnp
from jax.experimental import pallas as pl


def kernel(customer_ids, ticket_tokens, customer_table, subject_table, W1, b1, W2, b2):
    raise NotImplementedError("write your pallas kernel here")


# speedup vs baseline: 11.6722x; 11.6722x over previous
import functools

import jax
import jax.numpy as jnp
from jax import lax
from jax.experimental import pallas as pl
from jax.experimental.pallas import tpu as pltpu
from jax.experimental.pallas import tpu_sc as plsc

B = 16384
SEQ = 20
D = 32
NC = 2
NS = 16
NW = NC * NS
B_PER_W = B // NW
CHUNK = 128
NCHUNK = B_PER_W // CHUNK


def _sc_feat_kernel(cid_hbm, tok_hbm, ctab_hbm, stab_hbm, feat_hbm,
                    cid_v, tok_v, crows_v, trows_v, feat_v, sem):
    wid = lax.axis_index("s") * NC + lax.axis_index("c")

    def do_chunk(c, _):
        base = wid * B_PER_W + c * CHUNK
        pltpu.sync_copy(cid_hbm.at[pl.ds(base, CHUNK)], cid_v)
        pltpu.sync_copy(tok_hbm.at[pl.ds(base * SEQ, CHUNK * SEQ)], tok_v)
        copies = [pltpu.async_copy(ctab_hbm.at[cid_v], crows_v, sem)]
        for j in range(SEQ):
            copies.append(
                pltpu.async_copy(
                    stab_hbm.at[tok_v.at[pl.ds(j * CHUNK, CHUNK)]],
                    trows_v.at[pl.ds(j * CHUNK, CHUNK)],
                    sem,
                )
            )
        for cp in copies:
            cp.wait()

        def do_sample(i, _):
            for h in range(2):
                sl = pl.ds(h * 16, 16)
                acc = trows_v[i * SEQ, sl]
                for t in range(1, SEQ):
                    acc = acc + trows_v[i * SEQ + t, sl]
                feat_v[i, pl.ds(h * 16, 16)] = crows_v[i, sl]
                feat_v[i, pl.ds(32 + h * 16, 16)] = acc * (1.0 / SEQ)
            return 0

        lax.fori_loop(0, CHUNK, do_sample, 0)
        pltpu.sync_copy(feat_v, feat_hbm.at[pl.ds(base, CHUNK)])
        return 0

    lax.fori_loop(0, NCHUNK, do_chunk, 0)


def _build_feat(customer_ids, tokens_flat, customer_table, subject_table):
    mesh = plsc.VectorSubcoreMesh(
        core_axis_name="c", subcore_axis_name="s",
        num_cores=NC, num_subcores=NS,
    )
    return pl.kernel(
        _sc_feat_kernel,
        out_type=jax.ShapeDtypeStruct((B, 2 * D), jnp.float32),
        mesh=mesh,
        scratch_types=[
            pltpu.VMEM((CHUNK,), jnp.int32),
            pltpu.VMEM((CHUNK * SEQ,), jnp.int32),
            pltpu.VMEM((CHUNK, D), jnp.float32),
            pltpu.VMEM((CHUNK * SEQ, D), jnp.float32),
            pltpu.VMEM((CHUNK, 2 * D), jnp.float32),
            pltpu.SemaphoreType.DMA,
        ],
        compiler_params=pltpu.CompilerParams(use_tc_tiling_on_sc=False),
    )(customer_ids, tokens_flat, customer_table, subject_table)


def _mlp_body(feat_ref, w1_ref, b1_ref, w2_ref, b2_ref, out_ref):
    h = jnp.dot(feat_ref[...], w1_ref[...], preferred_element_type=jnp.float32)
    h = jnp.maximum(h + b1_ref[...], 0.0)
    o = jnp.dot(h, w2_ref[...], preferred_element_type=jnp.float32)
    out_ref[...] = o + b2_ref[...]


def _mlp(feat, W1, b1, W2, b2):
    blk = 2048
    grid = (B // blk,)
    return pl.pallas_call(
        _mlp_body,
        grid=grid,
        in_specs=[
            pl.BlockSpec((blk, 2 * D), lambda i: (i, 0)),
            pl.BlockSpec((2 * D, 64), lambda i: (0, 0)),
            pl.BlockSpec((64,), lambda i: (0,)),
            pl.BlockSpec((64, D), lambda i: (0, 0)),
            pl.BlockSpec((D,), lambda i: (0,)),
        ],
        out_specs=pl.BlockSpec((blk, D), lambda i: (i, 0)),
        out_shape=jax.ShapeDtypeStruct((B, D), jnp.float32),
    )(feat, W1, b1, W2, b2)


@jax.jit
def kernel(customer_ids, ticket_tokens, customer_table, subject_table, W1, b1, W2, b2):
    tokens_flat = ticket_tokens.reshape(-1).astype(jnp.int32)
    cids = customer_ids.astype(jnp.int32)
    feat = _build_feat(cids, tokens_flat, customer_table, subject_table)
    return _mlp(feat, W1, b1, W2, b2)


# speedup vs baseline: 14.1547x; 1.2127x over previous
import functools

import jax
import jax.numpy as jnp
from jax import lax
from jax.experimental import pallas as pl
from jax.experimental.pallas import tpu as pltpu
from jax.experimental.pallas import tpu_sc as plsc

B = 16384
SEQ = 20
D = 32
CVOC = 100001
CVOC_P = 100008
SVOC = 10000
NC = 2
NS = 16
NW = NC * NS
CH_C = 2048
CH_S = 1024


def _sc_feat_kernel(cid_hbm, tok_hbm, ctab_hbm, stab_hbm, feat_hbm,
                    row_v, cid_v, tokc_v, outc_v, outs_v, sem):
    f = lax.axis_index("s") * NC + lax.axis_index("c")

    base_w = f * CVOC
    base8 = (base_w // 8) * 8
    shift = base_w - base8
    pltpu.sync_copy(ctab_hbm.at[pl.ds(base8, CVOC_P)], row_v)

    def cust_chunk(c, _):
        base = c * CH_C
        pltpu.sync_copy(cid_hbm.at[pl.ds(base, CH_C)], cid_v)

        def cust_group(j, _):
            idx = cid_v[pl.ds(j * 16, 16)] + shift
            outc_v[pl.ds(j * 16, 16)] = plsc.load_gather(row_v, [idx])
            return 0

        lax.fori_loop(0, CH_C // 16, cust_group, 0)
        pltpu.sync_copy(outc_v, feat_hbm.at[f, pl.ds(base, CH_C)])
        return 0

    lax.fori_loop(0, B // CH_C, cust_chunk, 0)

    pltpu.sync_copy(stab_hbm.at[f], row_v.at[pl.ds(0, SVOC)])

    def subj_chunk(c, _):
        base = c * CH_S
        copies = [
            pltpu.async_copy(tok_hbm.at[t, pl.ds(base, CH_S)],
                             tokc_v.at[t], sem)
            for t in range(SEQ)
        ]
        for cp in copies:
            cp.wait()

        def subj_group(j, _):
            sl = pl.ds(j * 16, 16)
            acc = plsc.load_gather(row_v, [tokc_v[0, sl]])
            for t in range(1, SEQ):
                acc = acc + plsc.load_gather(row_v, [tokc_v[t, sl]])
            outs_v[sl] = acc * (1.0 / SEQ)
            return 0

        lax.fori_loop(0, CH_S // 16, subj_group, 0)
        pltpu.sync_copy(outs_v, feat_hbm.at[D + f, pl.ds(base, CH_S)])
        return 0

    lax.fori_loop(0, B // CH_S, subj_chunk, 0)


def _build_featT(customer_ids, tokT, ctab_flat, stabT):
    mesh = plsc.VectorSubcoreMesh(
        core_axis_name="c", subcore_axis_name="s",
        num_cores=NC, num_subcores=NS,
    )
    return pl.kernel(
        _sc_feat_kernel,
        out_type=jax.ShapeDtypeStruct((2 * D, B), jnp.float32),
        mesh=mesh,
        scratch_types=[
            pltpu.VMEM((CVOC_P,), jnp.float32),
            pltpu.VMEM((CH_C,), jnp.int32),
            pltpu.VMEM((SEQ, CH_S), jnp.int32),
            pltpu.VMEM((CH_C,), jnp.float32),
            pltpu.VMEM((CH_S,), jnp.float32),
            pltpu.SemaphoreType.DMA,
        ],
        compiler_params=pltpu.CompilerParams(
            use_tc_tiling_on_sc=False, needs_layout_passes=False),
    )(customer_ids, tokT, ctab_flat, stabT)


def _mlp_body(feat_ref, w1t_ref, b1_ref, w2t_ref, b2_ref, out_ref):
    hT = lax.dot_general(w1t_ref[...], feat_ref[...],
                         (((1,), (0,)), ((), ())),
                         preferred_element_type=jnp.float32)
    hT = jnp.maximum(hT + b1_ref[...], 0.0)
    oT = lax.dot_general(w2t_ref[...], hT,
                         (((1,), (0,)), ((), ())),
                         preferred_element_type=jnp.float32)
    out_ref[...] = oT + b2_ref[...]


def _mlp_T(featT, W1T, b1c, W2T, b2c):
    blk = 4096
    return pl.pallas_call(
        _mlp_body,
        grid=(B // blk,),
        in_specs=[
            pl.BlockSpec((2 * D, blk), lambda i: (0, i)),
            pl.BlockSpec((2 * D, 2 * D), lambda i: (0, 0)),
            pl.BlockSpec((2 * D, 1), lambda i: (0, 0)),
            pl.BlockSpec((D, 2 * D), lambda i: (0, 0)),
            pl.BlockSpec((D, 1), lambda i: (0, 0)),
        ],
        out_specs=pl.BlockSpec((D, blk), lambda i: (0, i)),
        out_shape=jax.ShapeDtypeStruct((D, B), jnp.float32),
    )(featT, W1T, b1c, W2T, b2c)


@jax.jit
def kernel(customer_ids, ticket_tokens, customer_table, subject_table, W1, b1, W2, b2):
    cids = customer_ids.astype(jnp.int32)
    tokT = ticket_tokens.astype(jnp.int32).T
    ctab_flat = customer_table.T.reshape(-1)
    stabT = subject_table.T
    featT = _build_featT(cids, tokT, ctab_flat, stabT)
    b1c = b1.reshape(2 * D, 1)
    b2c = b2.reshape(D, 1)
    outT = _mlp_T(featT, W1.T, b1c, W2.T, b2c)
    return outT.T


# speedup vs baseline: 16.2214x; 1.1460x over previous
import functools

import jax
import jax.numpy as jnp
from jax import lax
from jax.experimental import pallas as pl
from jax.experimental.pallas import tpu as pltpu
from jax.experimental.pallas import tpu_sc as plsc

B = 16384
SEQ = 20
D = 32
CVOC = 100001
CVOC_P = 100008
SVOC = 10000
NC = 2
NS = 16
CH_S = 2048
NCH_S = B // CH_S
CH_C = 4096
NCH_C = B // CH_C

_MESH = dict(core_axis_name="c", subcore_axis_name="s",
             num_cores=NC, num_subcores=NS)
_PARAMS = pltpu.CompilerParams(use_tc_tiling_on_sc=False,
                               needs_layout_passes=False)


def _sc_subj_kernel(tok_hbm, stab_hbm, feat_hbm,
                    row_v, tok2_v, out2_v, sem_t0, sem_t1, sem_o0, sem_o1):
    f = lax.axis_index("s") * NC + lax.axis_index("c")
    sem_t = (sem_t0, sem_t1)
    sem_o = (sem_o0, sem_o1)
    pltpu.sync_copy(stab_hbm.at[f], row_v)

    def fire(c):
        buf = c % 2
        for t in range(SEQ):
            pltpu.async_copy(tok_hbm.at[t, pl.ds(c * CH_S, CH_S)],
                             tok2_v.at[buf, t], sem_t[buf])

    def drain(c):
        buf = c % 2
        for t in range(SEQ):
            pltpu.make_async_copy(tok_hbm.at[t, pl.ds(c * CH_S, CH_S)],
                                  tok2_v.at[buf, t], sem_t[buf]).wait()

    fire(0)
    for c in range(NCH_S):
        if c + 1 < NCH_S:
            fire(c + 1)
        drain(c)
        buf = c % 2

        @plsc.parallel_loop(0, CH_S // 16, unroll=2)
        def subj_group(j):
            sl = pl.ds(j * 16, 16)
            acc = plsc.load_gather(row_v, [tok2_v[buf, 0, sl]])
            for t in range(1, SEQ):
                acc = acc + plsc.load_gather(row_v, [tok2_v[buf, t, sl]])
            out2_v[buf, sl] = acc * (1.0 / SEQ)

        if c >= 2:
            pltpu.make_async_copy(
                out2_v.at[buf],
                feat_hbm.at[f, pl.ds((c - 2) * CH_S, CH_S)],
                sem_o[buf]).wait()
        pltpu.async_copy(out2_v.at[buf],
                         feat_hbm.at[f, pl.ds(c * CH_S, CH_S)], sem_o[buf])
    for c in (NCH_S - 2, NCH_S - 1):
        buf = c % 2
        pltpu.make_async_copy(out2_v.at[buf],
                              feat_hbm.at[f, pl.ds(c * CH_S, CH_S)],
                              sem_o[buf]).wait()


def _sc_cust_kernel(cid_hbm, ctab_hbm, feat_hbm,
                    row_v, cid_v, out2_v, sem_o0, sem_o1):
    f = lax.axis_index("s") * NC + lax.axis_index("c")
    sem_o = (sem_o0, sem_o1)
    base_w = f * CVOC
    base8 = (base_w // 8) * 8
    shift = base_w - base8
    pltpu.sync_copy(ctab_hbm.at[pl.ds(base8, CVOC_P)], row_v)
    pltpu.sync_copy(cid_hbm, cid_v)

    for c in range(NCH_C):
        buf = c % 2

        @plsc.parallel_loop(0, CH_C // 16, unroll=4)
        def cust_group(j):
            sl = pl.ds(j * 16, 16)
            idx = cid_v[pl.ds(c * CH_C + j * 16, 16)] + shift
            out2_v[buf, sl] = plsc.load_gather(row_v, [idx])

        if c >= 2:
            pltpu.make_async_copy(
                out2_v.at[buf],
                feat_hbm.at[f, pl.ds((c - 2) * CH_C, CH_C)],
                sem_o[buf]).wait()
        pltpu.async_copy(out2_v.at[buf],
                         feat_hbm.at[f, pl.ds(c * CH_C, CH_C)], sem_o[buf])
    for c in (NCH_C - 2, NCH_C - 1):
        buf = c % 2
        pltpu.make_async_copy(out2_v.at[buf],
                              feat_hbm.at[f, pl.ds(c * CH_C, CH_C)],
                              sem_o[buf]).wait()


def _build_featS(tokT, stabT):
    return pl.kernel(
        _sc_subj_kernel,
        out_type=jax.ShapeDtypeStruct((D, B), jnp.float32),
        mesh=plsc.VectorSubcoreMesh(**_MESH),
        scratch_types=[
            pltpu.VMEM((SVOC,), jnp.float32),
            pltpu.VMEM((2, SEQ, CH_S), jnp.int32),
            pltpu.VMEM((2, CH_S), jnp.float32),
            pltpu.SemaphoreType.DMA,
            pltpu.SemaphoreType.DMA,
            pltpu.SemaphoreType.DMA,
            pltpu.SemaphoreType.DMA,
        ],
        compiler_params=_PARAMS,
    )(tokT, stabT)


def _build_featC(customer_ids, ctab_flat):
    return pl.kernel(
        _sc_cust_kernel,
        out_type=jax.ShapeDtypeStruct((D, B), jnp.float32),
        mesh=plsc.VectorSubcoreMesh(**_MESH),
        scratch_types=[
            pltpu.VMEM((CVOC_P,), jnp.float32),
            pltpu.VMEM((B,), jnp.int32),
            pltpu.VMEM((2, CH_C), jnp.float32),
            pltpu.SemaphoreType.DMA,
            pltpu.SemaphoreType.DMA,
        ],
        compiler_params=_PARAMS,
    )(customer_ids, ctab_flat)


def _mlp_body(featc_ref, feats_ref, w1t_ref, b1_ref, w2t_ref, b2_ref, out_ref):
    hT = lax.dot_general(w1t_ref[:, 0:D], featc_ref[...],
                         (((1,), (0,)), ((), ())),
                         preferred_element_type=jnp.float32)
    hT = hT + lax.dot_general(w1t_ref[:, D:2 * D], feats_ref[...],
                              (((1,), (0,)), ((), ())),
                              preferred_element_type=jnp.float32)
    hT = jnp.maximum(hT + b1_ref[...], 0.0)
    oT = lax.dot_general(w2t_ref[...], hT,
                         (((1,), (0,)), ((), ())),
                         preferred_element_type=jnp.float32)
    out_ref[...] = oT + b2_ref[...]


def _mlp_T(featC, featS, W1T, b1c, W2T, b2c):
    blk = 4096
    return pl.pallas_call(
        _mlp_body,
        grid=(B // blk,),
        in_specs=[
            pl.BlockSpec((D, blk), lambda i: (0, i)),
            pl.BlockSpec((D, blk), lambda i: (0, i)),
            pl.BlockSpec((2 * D, 2 * D), lambda i: (0, 0)),
            pl.BlockSpec((2 * D, 1), lambda i: (0, 0)),
            pl.BlockSpec((D, 2 * D), lambda i: (0, 0)),
            pl.BlockSpec((D, 1), lambda i: (0, 0)),
        ],
        out_specs=pl.BlockSpec((D, blk), lambda i: (0, i)),
        out_shape=jax.ShapeDtypeStruct((D, B), jnp.float32),
    )(featC, featS, W1T, b1c, W2T, b2c)


@jax.jit
def kernel(customer_ids, ticket_tokens, customer_table, subject_table, W1, b1, W2, b2):
    cids = customer_ids.astype(jnp.int32)
    tokT = ticket_tokens.astype(jnp.int32).T
    ctab_flat = customer_table.T.reshape(-1)
    stabT = subject_table.T
    featS = _build_featS(tokT, stabT)
    featC = _build_featC(cids, ctab_flat)
    b1c = b1.reshape(2 * D, 1)
    b2c = b2.reshape(D, 1)
    outT = _mlp_T(featC, featS, W1.T, b1c, W2.T, b2c)
    return outT.T


# speedup vs baseline: 25.6412x; 1.5807x over previous
import functools

import jax
import jax.numpy as jnp
from jax import lax
from jax.experimental import pallas as pl
from jax.experimental.pallas import tpu as pltpu
from jax.experimental.pallas import tpu_sc as plsc

B = 16384
SEQ = 20
D = 32
CVOC = 100001
CVOC_P = 100008
SVOC = 10000
NC = 2
NS = 16
NW = NC * NS
CH_S = 1024
CH_C = 4096
NCH_C = B // CH_C

_MESH = dict(core_axis_name="c", subcore_axis_name="s",
             num_cores=NC, num_subcores=NS)
_PARAMS = pltpu.CompilerParams(use_tc_tiling_on_sc=False,
                               needs_layout_passes=False)


G = 4
NFG = D // G
S_PER = B // (NW // NFG)
NCH_G = S_PER // CH_S


def _sc_subj_kernel(tok_hbm, stab_hbm, feat_hbm,
                    row_v, tok2_v, out2_v, sem_t0, sem_t1, sem_o0, sem_o1):
    w = lax.axis_index("s") * NC + lax.axis_index("c")
    fg = w % NFG
    sr = w // NFG
    s0 = sr * S_PER
    sem_t = (sem_t0, sem_t1)
    sem_o = (sem_o0, sem_o1)
    SRCH = 2000
    row_copies = []
    for k in range(G):
        for q in range(SVOC // SRCH):
            row_copies.append(pltpu.async_copy(
                stab_hbm.at[fg * G + k, pl.ds(q * SRCH, SRCH)],
                row_v.at[pl.ds(k * SVOC + q * SRCH, SRCH)], sem_o0))

    def fire(c):
        buf = c % 2
        for t in range(SEQ):
            pltpu.async_copy(tok_hbm.at[t, pl.ds(s0 + c * CH_S, CH_S)],
                             tok2_v.at[buf, t], sem_t[buf])

    def drain(c):
        buf = c % 2
        for t in range(SEQ):
            pltpu.make_async_copy(tok_hbm.at[t, pl.ds(s0 + c * CH_S, CH_S)],
                                  tok2_v.at[buf, t], sem_t[buf]).wait()

    fire(0)
    for cp in row_copies:
        cp.wait()
    for c in range(NCH_G):
        if c + 1 < NCH_G:
            fire(c + 1)
        drain(c)
        buf = c % 2
        if c >= 2:
            for k in range(G):
                pltpu.make_async_copy(
                    out2_v.at[buf, k],
                    feat_hbm.at[fg * G + k, pl.ds(s0 + (c - 2) * CH_S, CH_S)],
                    sem_o[buf]).wait()

        @plsc.parallel_loop(0, CH_S // 16)
        def subj_group(j):
            sl = pl.ds(j * 16, 16)
            idxs = [tok2_v[buf, t, sl] for t in range(SEQ)]
            for k in range(G):
                acc = plsc.load_gather(row_v, [idxs[0] + (k * SVOC)])
                for t in range(1, SEQ):
                    acc = acc + plsc.load_gather(row_v, [idxs[t] + (k * SVOC)])
                out2_v[buf, k, sl] = acc * (1.0 / SEQ)

        for k in range(G):
            pltpu.async_copy(
                out2_v.at[buf, k],
                feat_hbm.at[fg * G + k, pl.ds(s0 + c * CH_S, CH_S)],
                sem_o[buf])
    for c in range(max(0, NCH_G - 2), NCH_G):
        buf = c % 2
        for k in range(G):
            pltpu.make_async_copy(
                out2_v.at[buf, k],
                feat_hbm.at[fg * G + k, pl.ds(s0 + c * CH_S, CH_S)],
                sem_o[buf]).wait()


def _sc_cust_kernel(cid_hbm, ctab_hbm, dep_hbm, feat_hbm,
                    row_v, cid_v, out2_v, sem_o0, sem_o1):
    del dep_hbm
    f = lax.axis_index("s") * NC + lax.axis_index("c")
    sem_o = (sem_o0, sem_o1)
    base_w = f * CVOC
    base8 = (base_w // 8) * 8
    shift = base_w - base8
    RCH = 8192
    row_copies = []
    for k in range(CVOC_P // RCH):
        row_copies.append(pltpu.async_copy(
            ctab_hbm.at[pl.ds(base8 + k * RCH, RCH)],
            row_v.at[pl.ds(k * RCH, RCH)], sem_o0))
    tail = CVOC_P % RCH
    tail_base = (CVOC_P // RCH) * RCH
    row_copies.append(pltpu.async_copy(
        ctab_hbm.at[pl.ds(base8 + tail_base, tail)],
        row_v.at[pl.ds(tail_base, tail)], sem_o0))
    for k in range(2):
        row_copies.append(pltpu.async_copy(
            cid_hbm.at[pl.ds(k * (B // 2), B // 2)],
            cid_v.at[pl.ds(k * (B // 2), B // 2)], sem_o0))
    for cp in row_copies:
        cp.wait()

    for c in range(NCH_C):
        buf = c % 2
        if c >= 2:
            pltpu.make_async_copy(
                out2_v.at[buf],
                feat_hbm.at[f, pl.ds((c - 2) * CH_C, CH_C)],
                sem_o[buf]).wait()

        @plsc.parallel_loop(0, CH_C // 16, unroll=4)
        def cust_group(j):
            sl = pl.ds(j * 16, 16)
            idx = cid_v[pl.ds(c * CH_C + j * 16, 16)] + shift
            out2_v[buf, sl] = plsc.load_gather(row_v, [idx])

        pltpu.async_copy(out2_v.at[buf],
                         feat_hbm.at[f, pl.ds(c * CH_C, CH_C)], sem_o[buf])
    for c in (NCH_C - 2, NCH_C - 1):
        buf = c % 2
        pltpu.make_async_copy(out2_v.at[buf],
                              feat_hbm.at[f, pl.ds(c * CH_C, CH_C)],
                              sem_o[buf]).wait()


def _build_featS(tokT, stabT):
    return pl.kernel(
        _sc_subj_kernel,
        out_type=jax.ShapeDtypeStruct((D, B), jnp.float32),
        mesh=plsc.VectorSubcoreMesh(**_MESH),
        scratch_types=[
            pltpu.VMEM((G * SVOC,), jnp.float32),
            pltpu.VMEM((2, SEQ, CH_S), jnp.int32),
            pltpu.VMEM((2, G, CH_S), jnp.float32),
            pltpu.SemaphoreType.DMA,
            pltpu.SemaphoreType.DMA,
            pltpu.SemaphoreType.DMA,
            pltpu.SemaphoreType.DMA,
        ],
        compiler_params=_PARAMS,
    )(tokT, stabT)


def _build_featC(customer_ids, ctab_flat, featS):
    return pl.kernel(
        _sc_cust_kernel,
        out_type=jax.ShapeDtypeStruct((D, B), jnp.float32),
        mesh=plsc.VectorSubcoreMesh(**_MESH),
        scratch_types=[
            pltpu.VMEM((CVOC_P,), jnp.float32),
            pltpu.VMEM((B,), jnp.int32),
            pltpu.VMEM((2, CH_C), jnp.float32),
            pltpu.SemaphoreType.DMA,
            pltpu.SemaphoreType.DMA,
        ],
        compiler_params=_PARAMS,
    )(customer_ids, ctab_flat, featS)


def _mlp_body(featc_ref, feats_ref, w1t_ref, b1_ref, w2t_ref, b2_ref, out_ref):
    hT = lax.dot_general(w1t_ref[:, 0:D], featc_ref[...],
                         (((1,), (0,)), ((), ())),
                         preferred_element_type=jnp.float32)
    hT = hT + lax.dot_general(w1t_ref[:, D:2 * D], feats_ref[...],
                              (((1,), (0,)), ((), ())),
                              preferred_element_type=jnp.float32)
    hT = jnp.maximum(hT + b1_ref[...], 0.0)
    oT = lax.dot_general(w2t_ref[...], hT,
                         (((1,), (0,)), ((), ())),
                         preferred_element_type=jnp.float32)
    out_ref[...] = oT + b2_ref[...]


def _mlp_T(featC, featS, W1T, b1c, W2T, b2c):
    blk = 4096
    return pl.pallas_call(
        _mlp_body,
        grid=(B // blk,),
        in_specs=[
            pl.BlockSpec((D, blk), lambda i: (0, i)),
            pl.BlockSpec((D, blk), lambda i: (0, i)),
            pl.BlockSpec((2 * D, 2 * D), lambda i: (0, 0)),
            pl.BlockSpec((2 * D, 1), lambda i: (0, 0)),
            pl.BlockSpec((D, 2 * D), lambda i: (0, 0)),
            pl.BlockSpec((D, 1), lambda i: (0, 0)),
        ],
        out_specs=pl.BlockSpec((D, blk), lambda i: (0, i)),
        out_shape=jax.ShapeDtypeStruct((D, B), jnp.float32),
    )(featC, featS, W1T, b1c, W2T, b2c)


@jax.jit
def kernel(customer_ids, ticket_tokens, customer_table, subject_table, W1, b1, W2, b2):
    cids = customer_ids.astype(jnp.int32)
    tokT = ticket_tokens.astype(jnp.int32).T
    ctab_flat = customer_table.T.reshape(-1)
    stabT = subject_table.T
    featS = _build_featS(tokT, stabT)
    featC = _build_featC(cids, ctab_flat, featS)
    b1c = b1.reshape(2 * D, 1)
    b2c = b2.reshape(D, 1)
    outT = _mlp_T(featC, featS, W1.T, b1c, W2.T, b2c)
    return outT.T
